# baseline (speedup 1.0000x reference)
import functools

import jax
import jax.numpy as jnp
from jax.experimental import pallas as pl
from jax.experimental.pallas import tpu as pltpu

NEG_FILL = -1000000000.0
K = 8
CHUNK = 16000
UNROLL = 40


def _erf(z):
    s = jnp.sign(z)
    a = jnp.abs(z)
    t = 1.0 / (1.0 + 0.3275911 * a)
    poly = t * (0.254829592 + t * (-0.284496736 + t * (1.421413741
           + t * (-1.453152027 + t * 1.061405429))))
    return s * (1.0 - poly * jnp.exp(-a * a))


def _fused_body(row_ref, col_ref, x_ref, w_ref, b_ref, out_ref, *aggr_refs,
                chunk, nsteps):
    step = pl.program_id(0)

    @pl.when(step == 0)
    def _init():
        for k in range(K):
            aggr_refs[k][...] = jnp.full_like(aggr_refs[k][...], NEG_FILL)

    def body(i, carry):
        for k in range(K):
            r = row_ref[0, 0, i * K + k]
            c = col_ref[0, 0, i * K + k]
            xr = x_ref[c, :]
            aggr_refs[k][r, :] = jnp.maximum(aggr_refs[k][r, :], xr)
        return carry

    jax.lax.fori_loop(0, chunk // K, body, 0, unroll=UNROLL)

    @pl.when(step == nsteps - 1)
    def _epilogue():
        a = aggr_refs[0][...]
        for k in range(1, K):
            a = jnp.maximum(a, aggr_refs[k][...])
        a = jnp.where(a == NEG_FILL, 0.0, a) - x_ref[...]
        z = jnp.dot(a, w_ref[...],
                    preferred_element_type=jnp.float32) + b_ref[...]
        out_ref[...] = 0.5 * z * (1.0 + _erf(z * 0.7071067811865476))


def kernel(x, edge_index, W, b):
    Bn, N, C = x.shape
    x_flat = x.reshape(N, C)
    E = edge_index.shape[1]
    nb = E // CHUNK
    row = edge_index[0].reshape(nb, 1, CHUNK)
    col = edge_index[1].reshape(nb, 1, CHUNK)

    out = pl.pallas_call(
        functools.partial(_fused_body, chunk=CHUNK, nsteps=nb),
        grid=(nb,),
        in_specs=[
            pl.BlockSpec((1, 1, CHUNK), lambda i: (i, 0, 0),
                         memory_space=pltpu.SMEM),
            pl.BlockSpec((1, 1, CHUNK), lambda i: (i, 0, 0),
                         memory_space=pltpu.SMEM),
            pl.BlockSpec((N, C), lambda i: (0, 0)),
            pl.BlockSpec((C, C), lambda i: (0, 0)),
            pl.BlockSpec((1, C), lambda i: (0, 0)),
        ],
        out_specs=pl.BlockSpec((N, C), lambda i: (0, 0)),
        out_shape=jax.ShapeDtypeStruct((N, C), jnp.float32),
        scratch_shapes=[pltpu.VMEM((N, C), jnp.float32) for _ in range(K)],
        compiler_params=pltpu.CompilerParams(
            dimension_semantics=("arbitrary",)),
    )(row, col, x_flat, W, b.reshape(1, C))
    return out.reshape(Bn, N, C)


# speedup vs baseline: 1.0277x; 1.0277x over previous
import functools

import jax
import jax.numpy as jnp
from jax.experimental import pallas as pl
from jax.experimental.pallas import tpu as pltpu

NEG_FILL = -1000000000.0
K = 8
CHUNK = 16000
UNROLL = 25


def _erf(z):
    s = jnp.sign(z)
    a = jnp.abs(z)
    t = 1.0 / (1.0 + 0.3275911 * a)
    poly = t * (0.254829592 + t * (-0.284496736 + t * (1.421413741
           + t * (-1.453152027 + t * 1.061405429))))
    return s * (1.0 - poly * jnp.exp(-a * a))


def _fused_body(row_ref, col_ref, x_ref, w_ref, b_ref, out_ref, *aggr_refs,
                chunk, nsteps):
    step = pl.program_id(0)

    @pl.when(step == 0)
    def _init():
        for k in range(K):
            aggr_refs[k][...] = jnp.full_like(aggr_refs[k][...], NEG_FILL)

    def body(i, carry):
        for k in range(K):
            r = row_ref[0, 0, i * K + k]
            c = col_ref[0, 0, i * K + k]
            xr = x_ref[c, :]
            aggr_refs[k][r, :] = jnp.maximum(aggr_refs[k][r, :], xr)
        return carry

    jax.lax.fori_loop(0, chunk // K, body, 0, unroll=UNROLL)

    @pl.when(step == nsteps - 1)
    def _epilogue():
        a = aggr_refs[0][...]
        for k in range(1, K):
            a = jnp.maximum(a, aggr_refs[k][...])
        a = jnp.where(a == NEG_FILL, 0.0, a) - x_ref[...]
        z = jnp.dot(a, w_ref[...],
                    preferred_element_type=jnp.float32) + b_ref[...]
        out_ref[...] = 0.5 * z * (1.0 + _erf(z * 0.7071067811865476))


def kernel(x, edge_index, W, b):
    Bn, N, C = x.shape
    x_flat = x.reshape(N, C)
    E = edge_index.shape[1]
    nb = E // CHUNK
    row = edge_index[0].reshape(nb, 1, CHUNK)
    col = edge_index[1].reshape(nb, 1, CHUNK)

    out = pl.pallas_call(
        functools.partial(_fused_body, chunk=CHUNK, nsteps=nb),
        grid=(nb,),
        in_specs=[
            pl.BlockSpec((1, 1, CHUNK), lambda i: (i, 0, 0),
                         memory_space=pltpu.SMEM),
            pl.BlockSpec((1, 1, CHUNK), lambda i: (i, 0, 0),
                         memory_space=pltpu.SMEM),
            pl.BlockSpec((N, C), lambda i: (0, 0)),
            pl.BlockSpec((C, C), lambda i: (0, 0)),
            pl.BlockSpec((1, C), lambda i: (0, 0)),
        ],
        out_specs=pl.BlockSpec((N, C), lambda i: (0, 0)),
        out_shape=jax.ShapeDtypeStruct((N, C), jnp.float32),
        scratch_shapes=[pltpu.VMEM((N, C), jnp.float32) for _ in range(K)],
        compiler_params=pltpu.CompilerParams(
            dimension_semantics=("arbitrary",)),
    )(row, col, x_flat, W, b.reshape(1, C))
    return out.reshape(Bn, N, C)
